# baseline (speedup 1.0000x reference)
import jax
import jax.numpy as jnp
from jax.experimental import pallas as pl

_S = 2
_S3 = _S * _S * _S


def _data_body(in_ref, out_ref):
    bn = in_ref.shape[0]
    c = in_ref.shape[1]
    out_ref[...] = jnp.broadcast_to(in_ref[...][:, None, :], (bn, _S3, c))


def _ijk_body(in_ref, out_ref):
    bn = in_ref.shape[0]
    kidx = jax.lax.broadcasted_iota(jnp.int32, (bn, _S3, 3), 1)
    cidx = jax.lax.broadcasted_iota(jnp.int32, (bn, _S3, 3), 2)
    off = jax.lax.shift_right_logical(kidx, 2 - cidx) & 1
    out_ref[...] = in_ref[...][:, None, :] * _S + off


def _joff_body(in_ref, out_ref):
    out_ref[...] = in_ref[...] * _S3


def kernel(coarse_data, coarse_ijk, joffsets):
    n, c = coarse_data.shape
    bn = 256
    fine3 = pl.pallas_call(
        _data_body,
        grid=(n // bn,),
        in_specs=[pl.BlockSpec((bn, c), lambda i: (i, 0))],
        out_specs=pl.BlockSpec((bn, _S3, c), lambda i: (i, 0, 0)),
        out_shape=jax.ShapeDtypeStruct((n, _S3, c), coarse_data.dtype),
    )(coarse_data)
    fine_data = fine3.reshape(n * _S3, c)

    bn2 = 2048
    ijk3 = pl.pallas_call(
        _ijk_body,
        grid=(n // bn2,),
        in_specs=[pl.BlockSpec((bn2, 3), lambda i: (i, 0))],
        out_specs=pl.BlockSpec((bn2, _S3, 3), lambda i: (i, 0, 0)),
        out_shape=jax.ShapeDtypeStruct((n, _S3, 3), coarse_ijk.dtype),
    )(coarse_ijk)
    fine_ijk = ijk3.reshape(n * _S3, 3)

    nj = joffsets.shape[0]
    joff2 = pl.pallas_call(
        _joff_body,
        in_specs=[pl.BlockSpec((1, nj), lambda: (0, 0))],
        out_specs=pl.BlockSpec((1, nj), lambda: (0, 0)),
        out_shape=jax.ShapeDtypeStruct((1, nj), joffsets.dtype),
    )(joffsets.reshape(1, nj))
    fine_joffsets = joff2.reshape(nj)

    return fine_data, fine_ijk, fine_joffsets


# speedup vs baseline: 1.2606x; 1.2606x over previous
import jax
import jax.numpy as jnp
from jax.experimental import pallas as pl

_S = 2
_S3 = _S * _S * _S


def _data_body(in_ref, out_ref):
    bn = in_ref.shape[0]
    c = in_ref.shape[1]
    out_ref[...] = jnp.broadcast_to(in_ref[...][:, None, :], (bn, _S3, c))


def _ijk_body(in_ref, out_ref):
    bn = in_ref.shape[0]
    kidx = jax.lax.broadcasted_iota(jnp.int32, (bn, _S3, 3), 1)
    cidx = jax.lax.broadcasted_iota(jnp.int32, (bn, _S3, 3), 2)
    off = jax.lax.shift_right_logical(kidx, 2 - cidx) & 1
    out_ref[...] = in_ref[...][:, None, :] * _S + off


def _joff_body(in_ref, out_ref):
    out_ref[...] = in_ref[...] * _S3


def kernel(coarse_data, coarse_ijk, joffsets):
    n, c = coarse_data.shape
    bn = 1024
    fine3 = pl.pallas_call(
        _data_body,
        grid=(n // bn,),
        in_specs=[pl.BlockSpec((bn, c), lambda i: (i, 0))],
        out_specs=pl.BlockSpec((bn, _S3, c), lambda i: (i, 0, 0)),
        out_shape=jax.ShapeDtypeStruct((n, _S3, c), coarse_data.dtype),
    )(coarse_data)
    fine_data = fine3.reshape(n * _S3, c)

    bn2 = 2048
    ijk3 = pl.pallas_call(
        _ijk_body,
        grid=(n // bn2,),
        in_specs=[pl.BlockSpec((bn2, 3), lambda i: (i, 0))],
        out_specs=pl.BlockSpec((bn2, _S3, 3), lambda i: (i, 0, 0)),
        out_shape=jax.ShapeDtypeStruct((n, _S3, 3), coarse_ijk.dtype),
    )(coarse_ijk)
    fine_ijk = ijk3.reshape(n * _S3, 3)

    nj = joffsets.shape[0]
    joff2 = pl.pallas_call(
        _joff_body,
        in_specs=[pl.BlockSpec((1, nj), lambda: (0, 0))],
        out_specs=pl.BlockSpec((1, nj), lambda: (0, 0)),
        out_shape=jax.ShapeDtypeStruct((1, nj), joffsets.dtype),
    )(joffsets.reshape(1, nj))
    fine_joffsets = joff2.reshape(nj)

    return fine_data, fine_ijk, fine_joffsets


# speedup vs baseline: 1.2862x; 1.0203x over previous
import jax
import jax.numpy as jnp
from jax.experimental import pallas as pl

_S = 2
_S3 = _S * _S * _S


def _data_body(in_ref, out_ref):
    bn = in_ref.shape[0]
    c = in_ref.shape[1]
    out_ref[...] = jnp.broadcast_to(in_ref[...][:, None, :], (bn, _S3, c))


def _ijk_body(in_ref, out_ref):
    bn = in_ref.shape[0]
    kidx = jax.lax.broadcasted_iota(jnp.int32, (bn, _S3, 3), 1)
    cidx = jax.lax.broadcasted_iota(jnp.int32, (bn, _S3, 3), 2)
    off = jax.lax.shift_right_logical(kidx, 2 - cidx) & 1
    out_ref[...] = in_ref[...][:, None, :] * _S + off


def _joff_body(in_ref, out_ref):
    out_ref[...] = in_ref[...] * _S3


def kernel(coarse_data, coarse_ijk, joffsets):
    n, c = coarse_data.shape
    bn = 2048
    fine3 = pl.pallas_call(
        _data_body,
        grid=(n // bn,),
        in_specs=[pl.BlockSpec((bn, c), lambda i: (i, 0))],
        out_specs=pl.BlockSpec((bn, _S3, c), lambda i: (i, 0, 0)),
        out_shape=jax.ShapeDtypeStruct((n, _S3, c), coarse_data.dtype),
    )(coarse_data)
    fine_data = fine3.reshape(n * _S3, c)

    bn2 = 2048
    ijk3 = pl.pallas_call(
        _ijk_body,
        grid=(n // bn2,),
        in_specs=[pl.BlockSpec((bn2, 3), lambda i: (i, 0))],
        out_specs=pl.BlockSpec((bn2, _S3, 3), lambda i: (i, 0, 0)),
        out_shape=jax.ShapeDtypeStruct((n, _S3, 3), coarse_ijk.dtype),
    )(coarse_ijk)
    fine_ijk = ijk3.reshape(n * _S3, 3)

    nj = joffsets.shape[0]
    joff2 = pl.pallas_call(
        _joff_body,
        in_specs=[pl.BlockSpec((1, nj), lambda: (0, 0))],
        out_specs=pl.BlockSpec((1, nj), lambda: (0, 0)),
        out_shape=jax.ShapeDtypeStruct((1, nj), joffsets.dtype),
    )(joffsets.reshape(1, nj))
    fine_joffsets = joff2.reshape(nj)

    return fine_data, fine_ijk, fine_joffsets


# speedup vs baseline: 1.3105x; 1.0189x over previous
import jax
import jax.numpy as jnp
from jax.experimental import pallas as pl
from jax.experimental.pallas import tpu as pltpu

_S = 2
_S3 = _S * _S * _S
_NBUF = 4


def _body(data_ref, ijk_ref, joff_ref, out_any, ijk_out, joff_out, dbuf, sems):
    i = pl.program_id(0)
    nsteps = pl.num_programs(0)
    bn = data_ref.shape[0]
    c = data_ref.shape[1]
    slot = jax.lax.rem(i, _NBUF)

    @pl.when(i >= _NBUF)
    def _wait_prev():
        pltpu.make_async_copy(
            dbuf.at[slot],
            out_any.at[pl.ds((i - _NBUF) * bn, bn)],
            sems.at[slot],
        ).wait()

    dbuf[slot] = jnp.broadcast_to(data_ref[...][:, None, :], (bn, _S3, c))
    pltpu.make_async_copy(
        dbuf.at[slot],
        out_any.at[pl.ds(i * bn, bn)],
        sems.at[slot],
    ).start()

    kidx = jax.lax.broadcasted_iota(jnp.int32, (bn, _S3, 3), 1)
    cidx = jax.lax.broadcasted_iota(jnp.int32, (bn, _S3, 3), 2)
    off = jax.lax.shift_right_logical(kidx, 2 - cidx) & 1
    ijk_out[...] = ijk_ref[...][:, None, :] * _S + off

    joff_out[...] = joff_ref[...] * _S3

    @pl.when(i == nsteps - 1)
    def _drain():
        for k in range(_NBUF):
            step = nsteps - _NBUF + k
            s = jax.lax.rem(step, _NBUF)
            pltpu.make_async_copy(
                dbuf.at[s],
                out_any.at[pl.ds(step * bn, bn)],
                sems.at[s],
            ).wait()


def kernel(coarse_data, coarse_ijk, joffsets):
    n, c = coarse_data.shape
    nj = joffsets.shape[0]
    bn = 1024
    grid = n // bn
    fine3, ijk3, joff2 = pl.pallas_call(
        _body,
        grid=(grid,),
        in_specs=[
            pl.BlockSpec((bn, c), lambda i: (i, 0)),
            pl.BlockSpec((bn, 3), lambda i: (i, 0)),
            pl.BlockSpec((1, nj), lambda i: (0, 0)),
        ],
        out_specs=[
            pl.BlockSpec(memory_space=pl.ANY),
            pl.BlockSpec((bn, _S3, 3), lambda i: (i, 0, 0)),
            pl.BlockSpec((1, nj), lambda i: (0, 0)),
        ],
        out_shape=[
            jax.ShapeDtypeStruct((n, _S3, c), coarse_data.dtype),
            jax.ShapeDtypeStruct((n, _S3, 3), coarse_ijk.dtype),
            jax.ShapeDtypeStruct((1, nj), joffsets.dtype),
        ],
        scratch_shapes=[
            pltpu.VMEM((_NBUF, bn, _S3, c), coarse_data.dtype),
            pltpu.SemaphoreType.DMA((_NBUF,)),
        ],
    )(coarse_data, coarse_ijk, joffsets.reshape(1, nj))
    return fine3.reshape(n * _S3, c), ijk3.reshape(n * _S3, 3), joff2.reshape(nj)


# speedup vs baseline: 1.9485x; 1.4868x over previous
import jax
import jax.numpy as jnp
from jax.experimental import pallas as pl
from jax.experimental.pallas import tpu as pltpu

_S = 2
_S3 = _S * _S * _S
_NBUF = 4


def _body(data_ref, out_any, dbuf, sems):
    i = pl.program_id(0)
    nsteps = pl.num_programs(0)
    bn = data_ref.shape[0]
    c = data_ref.shape[1]
    slot = jax.lax.rem(i, _NBUF)

    @pl.when(i >= _NBUF)
    def _wait_prev():
        pltpu.make_async_copy(
            dbuf.at[slot],
            out_any.at[pl.ds((i - _NBUF) * bn, bn)],
            sems.at[slot],
        ).wait()

    dbuf[slot] = jnp.broadcast_to(data_ref[...][:, None, :], (bn, _S3, c))
    pltpu.make_async_copy(
        dbuf.at[slot],
        out_any.at[pl.ds(i * bn, bn)],
        sems.at[slot],
    ).start()

    @pl.when(i == nsteps - 1)
    def _drain():
        for k in range(_NBUF):
            step = nsteps - _NBUF + k
            s = jax.lax.rem(step, _NBUF)
            pltpu.make_async_copy(
                dbuf.at[s],
                out_any.at[pl.ds(step * bn, bn)],
                sems.at[s],
            ).wait()


def kernel(coarse_data, coarse_ijk, joffsets):
    n, c = coarse_data.shape
    bn = 1024
    grid = n // bn
    fine3 = pl.pallas_call(
        _body,
        grid=(grid,),
        in_specs=[pl.BlockSpec((bn, c), lambda i: (i, 0))],
        out_specs=pl.BlockSpec(memory_space=pl.ANY),
        out_shape=jax.ShapeDtypeStruct((n, _S3, c), coarse_data.dtype),
        scratch_shapes=[
            pltpu.VMEM((_NBUF, bn, _S3, c), coarse_data.dtype),
            pltpu.SemaphoreType.DMA((_NBUF,)),
        ],
    )(coarse_data)
    r = jnp.arange(_S)
    offsets = jnp.stack(jnp.meshgrid(r, r, r, indexing="ij"), axis=-1).reshape(-1, 3)
    offsets = offsets.astype(coarse_ijk.dtype)
    fine_ijk = (coarse_ijk[:, None, :] * _S + offsets[None, :, :]).reshape(-1, 3)
    return fine3.reshape(n * _S3, c), fine_ijk, joffsets * _S3
